# Initial kernel scaffold; baseline (speedup 1.0000x reference)
import jax
import jax.numpy as jnp
from jax.experimental import pallas as pl


def kernel(features, voxel_indices, W_cls, b_cls, W_box, b_box, W_dir, b_dir):
    raise NotImplementedError("write your pallas kernel here")


# speedup vs baseline: 3.9206x; 3.9206x over previous
import numpy as np
import jax
import jax.numpy as jnp
from jax.experimental import pallas as pl

IN_CH = 128
FX = 216
FY = 248
X_STRIDE = 69.12 / (FX - 1)
Y_STRIDE = (39.68 * 2.0) / (FY - 1)
X0 = 0.0
Y0 = -39.68
Z_CENTER = -1.78 + 1.56 / 2.0
DXA, DYA, DZA = 3.9, 1.6, 1.56
DIAG = float(np.sqrt(DXA * DXA + DYA * DYA))
ROTATIONS = (0.0, 1.5707963267948966)
DIR_OFFSET = 0.78539
PERIOD = float(np.pi)

BLK = 2000


def _body(feat_ref, vi_ref, w_ref, b_ref, cls_ref, box_ref, bidx_ref):
    feat = feat_ref[...]
    preds = jnp.dot(feat, w_ref[...], preferred_element_type=jnp.float32)
    preds = preds + b_ref[...]
    vi = vi_ref[...].astype(jnp.float32)
    bcol = vi[:, 0:1]
    ya = vi[:, 1:2] * Y_STRIDE + Y0
    xa = vi[:, 2:3] * X_STRIDE + X0
    boxes = []
    for r, rot in enumerate(ROTATIONS):
        base = 2 + 7 * r
        xt = preds[:, base + 0:base + 1]
        yt = preds[:, base + 1:base + 2]
        zt = preds[:, base + 2:base + 3]
        dxt = preds[:, base + 3:base + 4]
        dyt = preds[:, base + 4:base + 5]
        dzt = preds[:, base + 5:base + 6]
        rt = preds[:, base + 6:base + 7]
        xg = xt * DIAG + xa
        yg = yt * DIAG + ya
        zg = zt * DZA + Z_CENTER
        dxg = jnp.exp(dxt) * DXA
        dyg = jnp.exp(dyt) * DYA
        dzg = jnp.exp(dzt) * DZA
        d0 = preds[:, 16 + 2 * r:17 + 2 * r]
        d1 = preds[:, 17 + 2 * r:18 + 2 * r]
        label = (d1 > d0).astype(jnp.float32)
        rshift = (rt + rot) - DIR_OFFSET
        dir_rot = rshift - jnp.floor(rshift / PERIOD) * PERIOD
        rg = dir_rot + DIR_OFFSET + PERIOD * label
        boxes.append(jnp.concatenate([xg, yg, zg, dxg, dyg, dzg, rg], axis=1))
    box_ref[...] = jnp.concatenate(boxes, axis=1)
    cls_ref[...] = preds[:, 0:2]
    bidx_ref[...] = jnp.concatenate([bcol, bcol], axis=1)


def kernel(features, voxel_indices, W_cls, b_cls, W_box, b_box, W_dir, b_dir):
    n = features.shape[0]
    W_all = jnp.concatenate([W_cls, W_box, W_dir], axis=1)
    b_all = jnp.concatenate([b_cls, b_box, b_dir]).reshape(1, 20)
    grid = n // BLK
    cls2, box14, bidx2 = pl.pallas_call(
        _body,
        grid=(grid,),
        in_specs=[
            pl.BlockSpec((BLK, IN_CH), lambda i: (i, 0)),
            pl.BlockSpec((BLK, 3), lambda i: (i, 0)),
            pl.BlockSpec((IN_CH, 20), lambda i: (0, 0)),
            pl.BlockSpec((1, 20), lambda i: (0, 0)),
        ],
        out_specs=[
            pl.BlockSpec((BLK, 2), lambda i: (i, 0)),
            pl.BlockSpec((BLK, 14), lambda i: (i, 0)),
            pl.BlockSpec((BLK, 2), lambda i: (i, 0)),
        ],
        out_shape=[
            jax.ShapeDtypeStruct((n, 2), jnp.float32),
            jax.ShapeDtypeStruct((n, 14), jnp.float32),
            jax.ShapeDtypeStruct((n, 2), jnp.float32),
        ],
    )(features, voxel_indices, W_all, b_all)
    return cls2.reshape(-1, 1), box14.reshape(-1, 7), bidx2.reshape(-1)


# speedup vs baseline: 5.5899x; 1.4258x over previous
import numpy as np
import jax
import jax.numpy as jnp
from jax import lax
from jax.experimental import pallas as pl

IN_CH = 128
FX = 216
FY = 248
X_STRIDE = 69.12 / (FX - 1)
Y_STRIDE = (39.68 * 2.0) / (FY - 1)
Y0 = -39.68
Z_CENTER = -1.78 + 1.56 / 2.0
DXA, DYA, DZA = 3.9, 1.6, 1.56
DIAG = float(np.sqrt(DXA * DXA + DYA * DYA))
ROTATIONS = (0.0, 1.5707963267948966)
DIR_OFFSET = 0.78539
PERIOD = float(np.pi)

BLK = 2048


def _body(feat_ref, vit_ref, w_ref, b_ref, cls_ref, box_ref, bidx_ref):
    p = lax.dot_general(w_ref[...], feat_ref[...], (((1,), (1,)), ((), ())),
                        preferred_element_type=jnp.float32)
    p = p + b_ref[...]
    vit = vit_ref[...]
    brow = vit[0:1, :]
    ya = vit[1:2, :] * Y_STRIDE
    xa = vit[2:3, :] * X_STRIDE

    def rfix(rrow, ddrow):
        v = rrow - DIR_OFFSET
        v = v - jnp.floor(v * (1.0 / PERIOD)) * PERIOD
        return v + DIR_OFFSET + PERIOD * (ddrow > 0.0).astype(jnp.float32)

    x0 = p[0:1, :] + xa
    y0 = p[1:2, :] + ya
    z0 = p[2:3, :]
    e0 = jnp.exp(p[3:6, :])
    r0 = rfix(p[6:7, :], p[14:15, :])
    x1 = p[7:8, :] + xa
    y1 = p[8:9, :] + ya
    z1 = p[9:10, :]
    e1 = jnp.exp(p[10:13, :])
    r1 = rfix(p[13:14, :], p[15:16, :])
    box_ref[...] = jnp.concatenate(
        [x0, y0, z0, e0, r0, x1, y1, z1, e1, r1], axis=0)
    cls_ref[...] = p[16:18, :]
    bidx_ref[...] = jnp.concatenate([brow, brow], axis=0)


def kernel(features, voxel_indices, W_cls, b_cls, W_box, b_box, W_dir, b_dir):
    n = features.shape[0]
    f32 = jnp.float32

    wb = W_box.T
    scale = jnp.array([DIAG, DIAG, DZA, 1.0, 1.0, 1.0, 1.0] * 2, f32)[:, None]
    badd = jnp.array(
        [0.0, Y0, Z_CENTER, np.log(DXA), np.log(DYA), np.log(DZA), ROTATIONS[0],
         0.0, Y0, Z_CENTER, np.log(DXA), np.log(DYA), np.log(DZA), ROTATIONS[1]],
        f32)
    w_box14 = wb * scale
    b_box14 = b_box * scale[:, 0] + badd
    w_dd = jnp.stack([W_dir[:, 1] - W_dir[:, 0], W_dir[:, 3] - W_dir[:, 2]], 0)
    b_dd = jnp.stack([b_dir[1] - b_dir[0], b_dir[3] - b_dir[2]])
    W18 = jnp.concatenate([w_box14, w_dd, W_cls.T], axis=0)
    b18 = jnp.concatenate([b_box14, b_dd, b_cls])[:, None]
    vit = voxel_indices.T.astype(f32)

    grid = pl.cdiv(n, BLK)
    cls_t, box_t, bidx_t = pl.pallas_call(
        _body,
        grid=(grid,),
        in_specs=[
            pl.BlockSpec((BLK, IN_CH), lambda i: (i, 0)),
            pl.BlockSpec((3, BLK), lambda i: (0, i)),
            pl.BlockSpec((18, IN_CH), lambda i: (0, 0)),
            pl.BlockSpec((18, 1), lambda i: (0, 0)),
        ],
        out_specs=[
            pl.BlockSpec((2, BLK), lambda i: (0, i)),
            pl.BlockSpec((14, BLK), lambda i: (0, i)),
            pl.BlockSpec((2, BLK), lambda i: (0, i)),
        ],
        out_shape=[
            jax.ShapeDtypeStruct((2, n), f32),
            jax.ShapeDtypeStruct((14, n), f32),
            jax.ShapeDtypeStruct((2, n), f32),
        ],
    )(features, vit, W18, b18)
    return (cls_t.T.reshape(-1, 1), box_t.T.reshape(-1, 7),
            bidx_t.T.reshape(-1))


# speedup vs baseline: 6.1856x; 1.1066x over previous
import numpy as np
import jax
import jax.numpy as jnp
from jax import lax
from jax.experimental import pallas as pl

IN_CH = 128
FX = 216
FY = 248
X_STRIDE = 69.12 / (FX - 1)
Y_STRIDE = (39.68 * 2.0) / (FY - 1)
Y0 = -39.68
Z_CENTER = -1.78 + 1.56 / 2.0
DXA, DYA, DZA = 3.9, 1.6, 1.56
DIAG = float(np.sqrt(DXA * DXA + DYA * DYA))
ROTATIONS = (0.0, 1.5707963267948966)
DIR_OFFSET = 0.78539
PERIOD = float(np.pi)

BLK = 2048


def _body(feat_ref, vit_ref, w_ref, b_ref, cls_ref, box_ref, bidx_ref):
    p = lax.dot_general(w_ref[...], feat_ref[...], (((1,), (1,)), ((), ())),
                        preferred_element_type=jnp.float32)
    p = p + b_ref[...]
    vit = vit_ref[...]
    brow = vit[0:1, :]
    ya = vit[1:2, :] * Y_STRIDE
    xa = vit[2:3, :] * X_STRIDE

    def rfix(rrow, ddrow):
        v = rrow - DIR_OFFSET
        v = v - jnp.floor(v * (1.0 / PERIOD)) * PERIOD
        return v + DIR_OFFSET + PERIOD * (ddrow > 0.0).astype(jnp.float32)

    x0 = p[0:1, :] + xa
    y0 = p[1:2, :] + ya
    z0 = p[2:3, :]
    e0 = jnp.exp(p[3:6, :])
    r0 = rfix(p[6:7, :], p[14:15, :])
    x1 = p[7:8, :] + xa
    y1 = p[8:9, :] + ya
    z1 = p[9:10, :]
    e1 = jnp.exp(p[10:13, :])
    r1 = rfix(p[13:14, :], p[15:16, :])
    box_t = jnp.concatenate(
        [x0, y0, z0, e0, r0, x1, y1, z1, e1, r1], axis=0)
    box_ref[...] = box_t.T
    cls_ref[...] = p[16:18, :].T
    bidx_ref[...] = jnp.concatenate([brow, brow], axis=0).T


def kernel(features, voxel_indices, W_cls, b_cls, W_box, b_box, W_dir, b_dir):
    n = features.shape[0]
    f32 = jnp.float32

    wb = W_box.T
    scale = jnp.array([DIAG, DIAG, DZA, 1.0, 1.0, 1.0, 1.0] * 2, f32)[:, None]
    badd = jnp.array(
        [0.0, Y0, Z_CENTER, np.log(DXA), np.log(DYA), np.log(DZA), ROTATIONS[0],
         0.0, Y0, Z_CENTER, np.log(DXA), np.log(DYA), np.log(DZA), ROTATIONS[1]],
        f32)
    w_box14 = wb * scale
    b_box14 = b_box * scale[:, 0] + badd
    w_dd = jnp.stack([W_dir[:, 1] - W_dir[:, 0], W_dir[:, 3] - W_dir[:, 2]], 0)
    b_dd = jnp.stack([b_dir[1] - b_dir[0], b_dir[3] - b_dir[2]])
    W18 = jnp.concatenate([w_box14, w_dd, W_cls.T], axis=0)
    b18 = jnp.concatenate([b_box14, b_dd, b_cls])[:, None]
    vit = voxel_indices.T.astype(f32)

    grid = pl.cdiv(n, BLK)
    cls_t, box_t, bidx_t = pl.pallas_call(
        _body,
        grid=(grid,),
        in_specs=[
            pl.BlockSpec((BLK, IN_CH), lambda i: (i, 0)),
            pl.BlockSpec((3, BLK), lambda i: (0, i)),
            pl.BlockSpec((18, IN_CH), lambda i: (0, 0)),
            pl.BlockSpec((18, 1), lambda i: (0, 0)),
        ],
        out_specs=[
            pl.BlockSpec((BLK, 2), lambda i: (i, 0)),
            pl.BlockSpec((BLK, 14), lambda i: (i, 0)),
            pl.BlockSpec((BLK, 2), lambda i: (i, 0)),
        ],
        out_shape=[
            jax.ShapeDtypeStruct((n, 2), f32),
            jax.ShapeDtypeStruct((n, 14), f32),
            jax.ShapeDtypeStruct((n, 2), f32),
        ],
    )(features, vit, W18, b18)
    return (cls_t.reshape(-1, 1), box_t.reshape(-1, 7), bidx_t.reshape(-1))
